# Initial kernel scaffold; baseline (speedup 1.0000x reference)
import jax
import jax.numpy as jnp
from jax.experimental import pallas as pl


def kernel(X_seq, edge_index_seq, W_in, b_in, W_g1, b_g1, W_g2, b_g2, W_ih, W_hh, b_ih, b_hh, W_att, b_att):
    raise NotImplementedError("write your pallas kernel here")


# speedup vs baseline: 7.4519x; 7.4519x over previous
import functools

import jax
import jax.numpy as jnp
from jax import lax
from jax.experimental import pallas as pl
from jax.experimental.pallas import tpu as pltpu
from jax.experimental.pallas import tpu_sc as plsc

_T, _N, _E, _D, _H = 4, 10000, 320000, 128, 128
_NC, _NS = 2, 16
_NW = _NC * _NS
_EPT = _E // _NW
_C = 80
_NCH = _EPT // _C
_NP = 10240
_RPS = _NP // _NS

_B = 1000
_G = _N // _B


@functools.lru_cache(maxsize=1)
def _sc_fns():
    mesh = plsc.VectorSubcoreMesh(
        core_axis_name="c", subcore_axis_name="s",
        num_cores=_NC, num_subcores=_NS)

    @functools.partial(
        pl.kernel,
        out_type=(jax.ShapeDtypeStruct((_T * _NP, _H), jnp.float32),
                  jax.ShapeDtypeStruct((_T * _NP, _H), jnp.float32)),
        mesh=mesh,
        scratch_types=[
            pltpu.VMEM((_C,), jnp.int32),
            pltpu.VMEM((_C,), jnp.int32),
            pltpu.VMEM((_C, _H), jnp.float32),
            pltpu.VMEM_SHARED((_NP, _H), jnp.float32),
            pltpu.SemaphoreType.DMA,
        ])
    def agg_fn(g_hbm, src_hbm, dst_hbm, zeros_hbm, out_a, out_b,
               src_v, dst_v, rows_v, acc_sh, sem):
        c = lax.axis_index("c")
        s = lax.axis_index("s")
        wid = c * _NS + s
        last = _N - (_NS - 1) * _RPS
        for t in range(_T):
            @pl.when((c == 0) & (s < _NS - 1))
            def _(t=t):
                pltpu.sync_copy(g_hbm.at[pl.ds(t * _N + s * _RPS, _RPS)],
                                acc_sh.at[pl.ds(s * _RPS, _RPS)])

            @pl.when((c == 0) & (s == _NS - 1))
            def _(t=t):
                pltpu.sync_copy(
                    g_hbm.at[pl.ds(t * _N + (_NS - 1) * _RPS, last)],
                    acc_sh.at[pl.ds((_NS - 1) * _RPS, last)])

            @pl.when(c == 1)
            def _():
                pltpu.sync_copy(zeros_hbm.at[pl.ds(s * _RPS, _RPS)],
                                acc_sh.at[pl.ds(s * _RPS, _RPS)])
            plsc.subcore_barrier()
            base = t * _E + wid * _EPT

            def body(i, carry, base=base):
                off = base + i * _C
                pltpu.sync_copy(src_hbm.at[pl.ds(off, _C)], src_v)
                pltpu.sync_copy(dst_hbm.at[pl.ds(off, _C)], dst_v)
                pltpu.async_copy(g_hbm.at[src_v], rows_v, sem).wait()
                pltpu.sync_copy(rows_v, acc_sh.at[dst_v], add=True)
                return carry

            lax.fori_loop(0, _NCH, body, 0)
            plsc.subcore_barrier()

            @pl.when(c == 0)
            def _(t=t):
                pltpu.sync_copy(acc_sh.at[pl.ds(s * _RPS, _RPS)],
                                out_a.at[pl.ds(t * _NP + s * _RPS, _RPS)])

            @pl.when(c == 1)
            def _(t=t):
                pltpu.sync_copy(acc_sh.at[pl.ds(s * _RPS, _RPS)],
                                out_b.at[pl.ds(t * _NP + s * _RPS, _RPS)])
            plsc.subcore_barrier()

    return agg_fn


def _nan_to_num(x):
    x = jnp.where(x != x, 0.0, x)
    x = jnp.where(x == jnp.inf, 5.0, x)
    x = jnp.where(x == -jnp.inf, -5.0, x)
    return x


def _k1(X_seq, W_in, b_in, W_g1, dinv_nt):
    def body(x_ref, win_ref, bin_ref, wg1_ref, dinv_ref, g1_ref):
        for t in range(_T):
            h = jnp.dot(x_ref[t], win_ref[...],
                        preferred_element_type=jnp.float32) + bin_ref[0]
            hw = jnp.dot(h, wg1_ref[...], preferred_element_type=jnp.float32)
            g1_ref[t] = hw * dinv_ref[:, t:t + 1]

    return pl.pallas_call(
        body,
        grid=(_G,),
        in_specs=[
            pl.BlockSpec((_T, _B, _D), lambda i: (0, i, 0)),
            pl.BlockSpec((_D, _H), lambda i: (0, 0)),
            pl.BlockSpec((1, _H), lambda i: (0, 0)),
            pl.BlockSpec((_H, _H), lambda i: (0, 0)),
            pl.BlockSpec((_B, _T), lambda i: (i, 0)),
        ],
        out_specs=pl.BlockSpec((_T, _B, _H), lambda i: (0, i, 0)),
        out_shape=jax.ShapeDtypeStruct((_T, _N, _H), jnp.float32),
    )(X_seq, W_in, b_in, W_g1, dinv_nt)


def _k2(acc_a, acc_b, dinv_nt, b_g1, W_g2):
    def body(a_ref, b_ref, dinv_ref, bg1_ref, wg2_ref, g2_ref):
        for t in range(_T):
            dv = dinv_ref[:, t:t + 1]
            h1 = jax.nn.relu(dv * (a_ref[t] + b_ref[t]) + bg1_ref[0])
            hw2 = jnp.dot(h1, wg2_ref[...], preferred_element_type=jnp.float32)
            g2_ref[t] = hw2 * dv

    return pl.pallas_call(
        body,
        grid=(_G,),
        in_specs=[
            pl.BlockSpec((_T, _B, _H), lambda i: (0, i, 0)),
            pl.BlockSpec((_T, _B, _H), lambda i: (0, i, 0)),
            pl.BlockSpec((_B, _T), lambda i: (i, 0)),
            pl.BlockSpec((1, _H), lambda i: (0, 0)),
            pl.BlockSpec((_H, _H), lambda i: (0, 0)),
        ],
        out_specs=pl.BlockSpec((_T, _B, _H), lambda i: (0, i, 0)),
        out_shape=jax.ShapeDtypeStruct((_T, _N, _H), jnp.float32),
    )(acc_a, acc_b, dinv_nt, b_g1, W_g2)


def _k3(acc_a, acc_b, dinv_nt, b_g2, W_ihT, b_ih, W_hhT, b_hh, W_attT, b_att):
    def body(a_ref, b_ref, dinv_ref, bg2_ref, wih_ref, bih_ref,
             whh_ref, bhh_ref, watt_ref, batt_ref, ht_ref, z_ref):
        hprev = jnp.zeros((_B, _H), dtype=jnp.float32)
        hs = []
        atts = []
        for t in range(_T):
            dv = dinv_ref[:, t:t + 1]
            hst = _nan_to_num(dv * (a_ref[t] + b_ref[t]) + bg2_ref[0])
            gi = jnp.dot(hst, wih_ref[...],
                         preferred_element_type=jnp.float32) + bih_ref[0]
            gh = jnp.dot(hprev, whh_ref[...],
                         preferred_element_type=jnp.float32) + bhh_ref[0]
            r = jax.nn.sigmoid(gi[:, 0:_H] + gh[:, 0:_H])
            z = jax.nn.sigmoid(gi[:, _H:2 * _H] + gh[:, _H:2 * _H])
            ng = jnp.tanh(gi[:, 2 * _H:3 * _H] + r * gh[:, 2 * _H:3 * _H])
            h = (1.0 - z) * ng + z * hprev
            ht_ref[t] = h
            att = jnp.sum(h * watt_ref[0][None, :], axis=1, keepdims=True)
            att = jnp.clip(att + batt_ref[0, 0], -10.0, 10.0)
            hs.append(h)
            atts.append(att)
            hprev = h
        m = atts[0]
        for t in range(1, _T):
            m = jnp.maximum(m, atts[t])
        es = [jnp.exp(a - m) for a in atts]
        den = es[0]
        for t in range(1, _T):
            den = den + es[t]
        zfin = hs[0] * (es[0] / den)
        for t in range(1, _T):
            zfin = zfin + hs[t] * (es[t] / den)
        z_ref[...] = _nan_to_num(zfin)

    return pl.pallas_call(
        body,
        grid=(_G,),
        in_specs=[
            pl.BlockSpec((_T, _B, _H), lambda i: (0, i, 0)),
            pl.BlockSpec((_T, _B, _H), lambda i: (0, i, 0)),
            pl.BlockSpec((_B, _T), lambda i: (i, 0)),
            pl.BlockSpec((1, _H), lambda i: (0, 0)),
            pl.BlockSpec((_H, 3 * _H), lambda i: (0, 0)),
            pl.BlockSpec((1, 3 * _H), lambda i: (0, 0)),
            pl.BlockSpec((_H, 3 * _H), lambda i: (0, 0)),
            pl.BlockSpec((1, 3 * _H), lambda i: (0, 0)),
            pl.BlockSpec((1, _H), lambda i: (0, 0)),
            pl.BlockSpec((1, 1), lambda i: (0, 0)),
        ],
        out_specs=(
            pl.BlockSpec((_T, _B, _H), lambda i: (0, i, 0)),
            pl.BlockSpec((_B, _H), lambda i: (i, 0)),
        ),
        out_shape=(
            jax.ShapeDtypeStruct((_T, _N, _H), jnp.float32),
            jax.ShapeDtypeStruct((_N, _H), jnp.float32),
        ),
    )(acc_a, acc_b, dinv_nt, b_g2, W_ihT, b_ih, W_hhT, b_hh, W_attT, b_att)


def kernel(X_seq, edge_index_seq, W_in, b_in, W_g1, b_g1, W_g2, b_g2,
           W_ih, W_hh, b_ih, b_hh, W_att, b_att):
    agg_fn = _sc_fns()

    src = edge_index_seq[:, 0, :]
    dst = edge_index_seq[:, 1, :]
    src_adj = (src + (jnp.arange(_T, dtype=src.dtype) * _N)[:, None]
               ).reshape(-1)
    dst_flat = dst.reshape(-1)

    zeros_nh = jnp.zeros((_NP, _H), jnp.float32)
    ones_tnh = jnp.ones((_T * _N, _H), jnp.float32)

    deg_a, deg_b = agg_fn(ones_tnh, src_adj, dst_flat, zeros_nh)
    deg = (deg_a.reshape(_T, _NP, _H)[:, :_N, 0]
           + deg_b.reshape(_T, _NP, _H)[:, :_N, 0])
    dinv_nt = jnp.transpose(lax.rsqrt(deg))

    g1 = _k1(X_seq, W_in, b_in.reshape(1, _H), W_g1, dinv_nt)
    acc1_a, acc1_b = agg_fn(g1.reshape(_T * _N, _H), src_adj, dst_flat,
                            zeros_nh)
    g2 = _k2(acc1_a.reshape(_T, _NP, _H), acc1_b.reshape(_T, _NP, _H),
             dinv_nt, b_g1.reshape(1, _H), W_g2)
    acc2_a, acc2_b = agg_fn(g2.reshape(_T * _N, _H), src_adj, dst_flat,
                            zeros_nh)
    h_temporal, zfin = _k3(
        acc2_a.reshape(_T, _NP, _H), acc2_b.reshape(_T, _NP, _H), dinv_nt,
        b_g2.reshape(1, _H), jnp.transpose(W_ih), b_ih.reshape(1, 3 * _H),
        jnp.transpose(W_hh), b_hh.reshape(1, 3 * _H), jnp.transpose(W_att),
        b_att.reshape(1, 1))
    return (h_temporal, zfin)


# speedup vs baseline: 20.6337x; 2.7689x over previous
import functools

import jax
import jax.numpy as jnp
from jax import lax
from jax.experimental import pallas as pl
from jax.experimental.pallas import tpu as pltpu
from jax.experimental.pallas import tpu_sc as plsc

_T, _N, _E, _D, _H = 4, 10000, 320000, 128, 128
_NC, _NS = 2, 16
_NW = _NC * _NS
_EPT = _E // _NW
_C = 80
_NCH = _EPT // _C
_NP = 10240
_RPS = _NP // _NS

_B = 1000
_G = _N // _B


_NV = _C // 16
_PAIRS = _NCH // 2
_DEG_RING = 4
_DEG_GRP = (_NCH - 1) // _DEG_RING


@functools.lru_cache(maxsize=1)
def _sc_fns():
    mesh = plsc.VectorSubcoreMesh(
        core_axis_name="c", subcore_axis_name="s",
        num_cores=_NC, num_subcores=_NS)

    @functools.partial(
        pl.kernel,
        out_type=(jax.ShapeDtypeStruct((_T * _NP, _H), jnp.float32),
                  jax.ShapeDtypeStruct((_T * _NP, _H), jnp.float32)),
        mesh=mesh,
        scratch_types=[
            pltpu.VMEM((_EPT,), jnp.int32),
            pltpu.VMEM((_EPT,), jnp.int32),
            pltpu.VMEM((_C,), jnp.int32),
            pltpu.VMEM((_C,), jnp.int32),
            pltpu.VMEM((_C,), jnp.int32),
            pltpu.VMEM((_C,), jnp.int32),
            pltpu.VMEM((_C, _H), jnp.float32),
            pltpu.VMEM((_C, _H), jnp.float32),
            pltpu.VMEM_SHARED((_NP, _H), jnp.float32),
            pltpu.SemaphoreType.DMA,
            pltpu.SemaphoreType.DMA,
        ])
    def agg_fn(g_hbm, src_hbm, dst_hbm, zeros_hbm, out_a, out_b,
               src_all, dst_all, sv0, sv1, dv0, dv1, rows0, rows1,
               acc_sh, sem0, sem1):
        c = lax.axis_index("c")
        s = lax.axis_index("s")
        wid = c * _NS + s
        last = _N - (_NS - 1) * _RPS

        def load_idx(ch, sv, dv):
            for j in range(_NV):
                sv[pl.ds(j * 16, 16)] = src_all[pl.ds(ch * _C + j * 16, 16)]
                dv[pl.ds(j * 16, 16)] = dst_all[pl.ds(ch * _C + j * 16, 16)]

        def gather(sv, rows, sem):
            return pltpu.async_copy(g_hbm.at[sv], rows, sem)

        def scat(rows, dv):
            pltpu.sync_copy(rows, acc_sh.at[dv], add=True)

        for t in range(_T):
            @pl.when((c == 0) & (s < _NS - 1))
            def _(t=t):
                pltpu.sync_copy(g_hbm.at[pl.ds(t * _N + s * _RPS, _RPS)],
                                acc_sh.at[pl.ds(s * _RPS, _RPS)])

            @pl.when((c == 0) & (s == _NS - 1))
            def _(t=t):
                pltpu.sync_copy(
                    g_hbm.at[pl.ds(t * _N + (_NS - 1) * _RPS, last)],
                    acc_sh.at[pl.ds((_NS - 1) * _RPS, last)])

            @pl.when(c == 1)
            def _():
                pltpu.sync_copy(zeros_hbm.at[pl.ds(s * _RPS, _RPS)],
                                acc_sh.at[pl.ds(s * _RPS, _RPS)])
            ebase = t * _E + wid * _EPT
            pltpu.sync_copy(src_hbm.at[pl.ds(ebase, _EPT)], src_all)
            pltpu.sync_copy(dst_hbm.at[pl.ds(ebase, _EPT)], dst_all)
            plsc.subcore_barrier()

            load_idx(0, sv0, dv0)
            gather(sv0, rows0, sem0).wait()

            def pair(p, carry):
                ch0 = 2 * p
                load_idx(ch0 + 1, sv1, dv1)
                d1 = gather(sv1, rows1, sem0)
                scat(rows0, dv0)
                d1.wait()
                load_idx(ch0 + 2, sv0, dv0)
                d0 = gather(sv0, rows0, sem0)
                scat(rows1, dv1)
                d0.wait()
                return carry

            lax.fori_loop(0, _PAIRS, pair, 0)
            scat(rows0, dv0)
            plsc.subcore_barrier()

            @pl.when(c == 0)
            def _(t=t):
                pltpu.sync_copy(acc_sh.at[pl.ds(s * _RPS, _RPS)],
                                out_a.at[pl.ds(t * _NP + s * _RPS, _RPS)])

            @pl.when(c == 1)
            def _(t=t):
                pltpu.sync_copy(acc_sh.at[pl.ds(s * _RPS, _RPS)],
                                out_b.at[pl.ds(t * _NP + s * _RPS, _RPS)])
            plsc.subcore_barrier()

    @functools.partial(
        pl.kernel,
        out_type=(jax.ShapeDtypeStruct((_T * _NP,), jnp.float32),
                  jax.ShapeDtypeStruct((_T * _NP,), jnp.float32)),
        mesh=mesh,
        scratch_types=[
            pltpu.VMEM((_EPT,), jnp.int32),
            pltpu.VMEM((_C,), jnp.int32),
            pltpu.VMEM((_C,), jnp.int32),
            pltpu.VMEM((_C,), jnp.int32),
            pltpu.VMEM((_C,), jnp.int32),
            pltpu.VMEM((_C,), jnp.float32),
            pltpu.VMEM_SHARED((_NP,), jnp.float32),
            pltpu.SemaphoreType.DMA,
        ])
    def deg_fn(dst_hbm, zeros_hbm, out_a, out_b,
               dst_all, dv0, dv1, dv2, dv3, ones_v, acc_sh, sem):
        c = lax.axis_index("c")
        s = lax.axis_index("s")
        wid = c * _NS + s
        dvs = (dv0, dv1, dv2, dv3)
        for j in range(_NV):
            ones_v[pl.ds(j * 16, 16)] = jnp.ones((16,), jnp.float32)

        def load_dst(ch, dv):
            for j in range(_NV):
                dv[pl.ds(j * 16, 16)] = dst_all[pl.ds(ch * _C + j * 16, 16)]

        for t in range(_T):
            pltpu.sync_copy(zeros_hbm.at[pl.ds(s * _RPS, _RPS)],
                            acc_sh.at[pl.ds(s * _RPS, _RPS)])
            ebase = t * _E + wid * _EPT
            pltpu.sync_copy(dst_hbm.at[pl.ds(ebase, _EPT)], dst_all)
            plsc.subcore_barrier()

            def group(q, carry):
                base = q * _DEG_RING
                ds_ = []
                for j in range(_DEG_RING):
                    load_dst(base + j, dvs[j])
                    ds_.append(pltpu.async_copy(ones_v, acc_sh.at[dvs[j]],
                                                sem, add=True))
                for d in ds_:
                    d.wait()
                return carry

            lax.fori_loop(0, _DEG_GRP, group, 0)
            load_dst(_NCH - 1, dv0)
            pltpu.sync_copy(ones_v, acc_sh.at[dv0], add=True)
            plsc.subcore_barrier()

            @pl.when(c == 0)
            def _(t=t):
                pltpu.sync_copy(acc_sh.at[pl.ds(s * _RPS, _RPS)],
                                out_a.at[pl.ds(t * _NP + s * _RPS, _RPS)])

            @pl.when(c == 1)
            def _(t=t):
                pltpu.sync_copy(acc_sh.at[pl.ds(s * _RPS, _RPS)],
                                out_b.at[pl.ds(t * _NP + s * _RPS, _RPS)])
            plsc.subcore_barrier()

    return agg_fn, deg_fn


def _nan_to_num(x):
    x = jnp.where(x != x, 0.0, x)
    x = jnp.where(x == jnp.inf, 5.0, x)
    x = jnp.where(x == -jnp.inf, -5.0, x)
    return x


def _k1(X_seq, W_in, b_in, W_g1, dinv_nt):
    def body(x_ref, win_ref, bin_ref, wg1_ref, dinv_ref, g1_ref):
        for t in range(_T):
            h = jnp.dot(x_ref[t], win_ref[...],
                        preferred_element_type=jnp.float32) + bin_ref[0]
            hw = jnp.dot(h, wg1_ref[...], preferred_element_type=jnp.float32)
            g1_ref[t] = hw * dinv_ref[:, t:t + 1]

    return pl.pallas_call(
        body,
        grid=(_G,),
        in_specs=[
            pl.BlockSpec((_T, _B, _D), lambda i: (0, i, 0)),
            pl.BlockSpec((_D, _H), lambda i: (0, 0)),
            pl.BlockSpec((1, _H), lambda i: (0, 0)),
            pl.BlockSpec((_H, _H), lambda i: (0, 0)),
            pl.BlockSpec((_B, _T), lambda i: (i, 0)),
        ],
        out_specs=pl.BlockSpec((_T, _B, _H), lambda i: (0, i, 0)),
        out_shape=jax.ShapeDtypeStruct((_T, _N, _H), jnp.float32),
    )(X_seq, W_in, b_in, W_g1, dinv_nt)


def _k2(acc_a, acc_b, dinv_nt, b_g1, W_g2):
    def body(a_ref, b_ref, dinv_ref, bg1_ref, wg2_ref, g2_ref):
        for t in range(_T):
            dv = dinv_ref[:, t:t + 1]
            h1 = jax.nn.relu(dv * (a_ref[t] + b_ref[t]) + bg1_ref[0])
            hw2 = jnp.dot(h1, wg2_ref[...], preferred_element_type=jnp.float32)
            g2_ref[t] = hw2 * dv

    return pl.pallas_call(
        body,
        grid=(_G,),
        in_specs=[
            pl.BlockSpec((_T, _B, _H), lambda i: (0, i, 0)),
            pl.BlockSpec((_T, _B, _H), lambda i: (0, i, 0)),
            pl.BlockSpec((_B, _T), lambda i: (i, 0)),
            pl.BlockSpec((1, _H), lambda i: (0, 0)),
            pl.BlockSpec((_H, _H), lambda i: (0, 0)),
        ],
        out_specs=pl.BlockSpec((_T, _B, _H), lambda i: (0, i, 0)),
        out_shape=jax.ShapeDtypeStruct((_T, _N, _H), jnp.float32),
    )(acc_a, acc_b, dinv_nt, b_g1, W_g2)


def _k3(acc_a, acc_b, dinv_nt, b_g2, W_ihT, b_ih, W_hhT, b_hh, W_attT, b_att):
    def body(a_ref, b_ref, dinv_ref, bg2_ref, wih_ref, bih_ref,
             whh_ref, bhh_ref, watt_ref, batt_ref, ht_ref, z_ref):
        hprev = jnp.zeros((_B, _H), dtype=jnp.float32)
        hs = []
        atts = []
        for t in range(_T):
            dv = dinv_ref[:, t:t + 1]
            hst = _nan_to_num(dv * (a_ref[t] + b_ref[t]) + bg2_ref[0])
            gi = jnp.dot(hst, wih_ref[...],
                         preferred_element_type=jnp.float32) + bih_ref[0]
            gh = jnp.dot(hprev, whh_ref[...],
                         preferred_element_type=jnp.float32) + bhh_ref[0]
            r = jax.nn.sigmoid(gi[:, 0:_H] + gh[:, 0:_H])
            z = jax.nn.sigmoid(gi[:, _H:2 * _H] + gh[:, _H:2 * _H])
            ng = jnp.tanh(gi[:, 2 * _H:3 * _H] + r * gh[:, 2 * _H:3 * _H])
            h = (1.0 - z) * ng + z * hprev
            ht_ref[t] = h
            att = jnp.sum(h * watt_ref[0][None, :], axis=1, keepdims=True)
            att = jnp.clip(att + batt_ref[0, 0], -10.0, 10.0)
            hs.append(h)
            atts.append(att)
            hprev = h
        m = atts[0]
        for t in range(1, _T):
            m = jnp.maximum(m, atts[t])
        es = [jnp.exp(a - m) for a in atts]
        den = es[0]
        for t in range(1, _T):
            den = den + es[t]
        zfin = hs[0] * (es[0] / den)
        for t in range(1, _T):
            zfin = zfin + hs[t] * (es[t] / den)
        z_ref[...] = _nan_to_num(zfin)

    return pl.pallas_call(
        body,
        grid=(_G,),
        in_specs=[
            pl.BlockSpec((_T, _B, _H), lambda i: (0, i, 0)),
            pl.BlockSpec((_T, _B, _H), lambda i: (0, i, 0)),
            pl.BlockSpec((_B, _T), lambda i: (i, 0)),
            pl.BlockSpec((1, _H), lambda i: (0, 0)),
            pl.BlockSpec((_H, 3 * _H), lambda i: (0, 0)),
            pl.BlockSpec((1, 3 * _H), lambda i: (0, 0)),
            pl.BlockSpec((_H, 3 * _H), lambda i: (0, 0)),
            pl.BlockSpec((1, 3 * _H), lambda i: (0, 0)),
            pl.BlockSpec((1, _H), lambda i: (0, 0)),
            pl.BlockSpec((1, 1), lambda i: (0, 0)),
        ],
        out_specs=(
            pl.BlockSpec((_T, _B, _H), lambda i: (0, i, 0)),
            pl.BlockSpec((_B, _H), lambda i: (i, 0)),
        ),
        out_shape=(
            jax.ShapeDtypeStruct((_T, _N, _H), jnp.float32),
            jax.ShapeDtypeStruct((_N, _H), jnp.float32),
        ),
    )(acc_a, acc_b, dinv_nt, b_g2, W_ihT, b_ih, W_hhT, b_hh, W_attT, b_att)


def kernel(X_seq, edge_index_seq, W_in, b_in, W_g1, b_g1, W_g2, b_g2,
           W_ih, W_hh, b_ih, b_hh, W_att, b_att):
    agg_fn, deg_fn = _sc_fns()

    src = edge_index_seq[:, 0, :]
    dst = edge_index_seq[:, 1, :]
    src_adj = (src + (jnp.arange(_T, dtype=src.dtype) * _N)[:, None]
               ).reshape(-1)
    dst_flat = dst.reshape(-1)

    zeros_nh = jnp.zeros((_NP, _H), jnp.float32)
    zeros_1d = jnp.zeros((_NP,), jnp.float32)

    deg_a, deg_b = deg_fn(dst_flat, zeros_1d)
    deg = (deg_a.reshape(_T, _NP)[:, :_N]
           + deg_b.reshape(_T, _NP)[:, :_N] + 1.0)
    dinv_nt = jnp.transpose(lax.rsqrt(deg))

    g1 = _k1(X_seq, W_in, b_in.reshape(1, _H), W_g1, dinv_nt)
    acc1_a, acc1_b = agg_fn(g1.reshape(_T * _N, _H), src_adj, dst_flat,
                            zeros_nh)
    g2 = _k2(acc1_a.reshape(_T, _NP, _H), acc1_b.reshape(_T, _NP, _H),
             dinv_nt, b_g1.reshape(1, _H), W_g2)
    acc2_a, acc2_b = agg_fn(g2.reshape(_T * _N, _H), src_adj, dst_flat,
                            zeros_nh)
    h_temporal, zfin = _k3(
        acc2_a.reshape(_T, _NP, _H), acc2_b.reshape(_T, _NP, _H), dinv_nt,
        b_g2.reshape(1, _H), jnp.transpose(W_ih), b_ih.reshape(1, 3 * _H),
        jnp.transpose(W_hh), b_hh.reshape(1, 3 * _H), jnp.transpose(W_att),
        b_att.reshape(1, 1))
    return (h_temporal, zfin)
